# baseline (speedup 1.0000x reference)
import jax
import jax.numpy as jnp
from jax.experimental import pallas as pl
from jax.experimental.pallas import tpu as pltpu

F32 = jnp.float32
BF16 = jnp.bfloat16


def _prep_kernel(idx_ref, ew_ref, c_ref, g_ref, u_ref, m_ref):
    idx = idx_ref[0]
    g = g_ref[idx]
    c = c_ref[...]
    ew = ew_ref[...]
    w = jnp.dot(ew, c, preferred_element_type=F32)
    p_dim, r, _ = g.shape
    m0 = jnp.zeros((r, r), dtype=F32)
    for p in range(p_dim):
        onehot = (jax.lax.broadcasted_iota(jnp.int32, (p_dim, 1), 0) == p)
        wp = jnp.dot(w, onehot.astype(F32), preferred_element_type=F32)
        m0 = m0 + wp * g[p]
    gn2 = jnp.sum(g * g)
    cn2 = jnp.sum(c * c)
    un2 = jnp.sum(u_ref[...] * u_ref[...])
    scale = jax.lax.rsqrt(gn2) * jax.lax.rsqrt(cn2) / un2
    m_ref[...] = m0 * scale


def _weff_kernel(w_ref, ui_ref, uall_ref, m_ref, o_ref):
    a = jnp.dot(ui_ref[...], m_ref[...], preferred_element_type=F32)
    adapt = jax.lax.dot_general(
        a, uall_ref[...], (((1,), (1,)), ((), ())),
        preferred_element_type=F32)
    o_ref[...] = (w_ref[...] + adapt).astype(BF16)


def _main_kernel(x_ref, w_ref, b_ref, o_ref):
    k = pl.program_id(2)

    @pl.when(k == 0)
    def _init():
        o_ref[...] = jnp.broadcast_to(b_ref[...], o_ref.shape)

    xb = x_ref[...].astype(BF16)
    o_ref[...] += jnp.dot(xb, w_ref[...], preferred_element_type=F32)


def kernel(x, tensor_idx, expert_weights, W, b, G, C, U):
    n, d_in = x.shape
    d_out = W.shape[1]
    k_dim, p_dim, r, _ = G.shape
    t_dim = expert_weights.shape[0]

    idx = jnp.asarray(tensor_idx, jnp.int32).reshape((1,))
    ew2 = expert_weights.reshape(1, t_dim).astype(F32)

    m_eff = pl.pallas_call(
        _prep_kernel,
        grid_spec=pltpu.PrefetchScalarGridSpec(
            num_scalar_prefetch=1,
            grid=(1,),
            in_specs=[
                pl.BlockSpec((1, t_dim), lambda i, idx_ref: (0, 0)),
                pl.BlockSpec((t_dim, p_dim), lambda i, idx_ref: (0, 0)),
                pl.BlockSpec((k_dim, p_dim, r, r), lambda i, idx_ref: (0, 0, 0, 0)),
                pl.BlockSpec((d_in, r), lambda i, idx_ref: (0, 0)),
            ],
            out_specs=pl.BlockSpec((r, r), lambda i, idx_ref: (0, 0)),
        ),
        out_shape=jax.ShapeDtypeStruct((r, r), F32),
    )(idx, ew2, C, G, U)

    bw = 1024
    w_eff = pl.pallas_call(
        _weff_kernel,
        grid=(d_in // bw,),
        in_specs=[
            pl.BlockSpec((bw, d_out), lambda i: (i, 0)),
            pl.BlockSpec((bw, r), lambda i: (i, 0)),
            pl.BlockSpec((d_out, r), lambda i: (0, 0)),
            pl.BlockSpec((r, r), lambda i: (0, 0)),
        ],
        out_specs=pl.BlockSpec((bw, d_out), lambda i: (i, 0)),
        out_shape=jax.ShapeDtypeStruct((d_in, d_out), BF16),
        compiler_params=pltpu.CompilerParams(
            dimension_semantics=("parallel",)),
    )(W, U, U, m_eff)

    bn, bj, bk = 1024, 2048, 1024
    b2 = b.reshape(1, d_out)
    out = pl.pallas_call(
        _main_kernel,
        grid=(n // bn, d_out // bj, d_in // bk),
        in_specs=[
            pl.BlockSpec((bn, bk), lambda i, j, k: (i, k)),
            pl.BlockSpec((bk, bj), lambda i, j, k: (k, j)),
            pl.BlockSpec((1, bj), lambda i, j, k: (0, j)),
        ],
        out_specs=pl.BlockSpec((bn, bj), lambda i, j, k: (i, j)),
        out_shape=jax.ShapeDtypeStruct((n, d_out), F32),
        compiler_params=pltpu.CompilerParams(
            dimension_semantics=("parallel", "parallel", "arbitrary")),
    )(x, w_eff, b2)
    return out


# speedup vs baseline: 1.0018x; 1.0018x over previous
import jax
import jax.numpy as jnp
from jax.experimental import pallas as pl
from jax.experimental.pallas import tpu as pltpu

F32 = jnp.float32
BF16 = jnp.bfloat16


def _prep_kernel(idx_ref, ew_ref, c_ref, g_ref, u_ref, m_ref):
    idx = idx_ref[0]
    g = g_ref[idx]
    c = c_ref[...]
    ew = ew_ref[...]
    w = jnp.dot(ew, c, preferred_element_type=F32)
    p_dim, r, _ = g.shape
    m0 = jnp.zeros((r, r), dtype=F32)
    for p in range(p_dim):
        onehot = (jax.lax.broadcasted_iota(jnp.int32, (p_dim, 1), 0) == p)
        wp = jnp.dot(w, onehot.astype(F32), preferred_element_type=F32)
        m0 = m0 + wp * g[p]
    gn2 = jnp.sum(g * g)
    cn2 = jnp.sum(c * c)
    un2 = jnp.sum(u_ref[...] * u_ref[...])
    scale = jax.lax.rsqrt(gn2) * jax.lax.rsqrt(cn2) / un2
    m_ref[...] = m0 * scale


def _weff_kernel(w_ref, ui_ref, uall_ref, m_ref, o_ref):
    a = jnp.dot(ui_ref[...], m_ref[...], preferred_element_type=F32)
    adapt = jax.lax.dot_general(
        a, uall_ref[...], (((1,), (1,)), ((), ())),
        preferred_element_type=F32)
    o_ref[...] = (w_ref[...] + adapt).astype(BF16)


def _main_kernel(x_ref, w_ref, b_ref, o_ref):
    xb = x_ref[...].astype(BF16)
    o_ref[...] = (jnp.dot(xb, w_ref[...], preferred_element_type=F32)
                  + b_ref[...])


def kernel(x, tensor_idx, expert_weights, W, b, G, C, U):
    n, d_in = x.shape
    d_out = W.shape[1]
    k_dim, p_dim, r, _ = G.shape
    t_dim = expert_weights.shape[0]

    idx = jnp.asarray(tensor_idx, jnp.int32).reshape((1,))
    ew2 = expert_weights.reshape(1, t_dim).astype(F32)

    m_eff = pl.pallas_call(
        _prep_kernel,
        grid_spec=pltpu.PrefetchScalarGridSpec(
            num_scalar_prefetch=1,
            grid=(1,),
            in_specs=[
                pl.BlockSpec((1, t_dim), lambda i, idx_ref: (0, 0)),
                pl.BlockSpec((t_dim, p_dim), lambda i, idx_ref: (0, 0)),
                pl.BlockSpec((k_dim, p_dim, r, r), lambda i, idx_ref: (0, 0, 0, 0)),
                pl.BlockSpec((d_in, r), lambda i, idx_ref: (0, 0)),
            ],
            out_specs=pl.BlockSpec((r, r), lambda i, idx_ref: (0, 0)),
        ),
        out_shape=jax.ShapeDtypeStruct((r, r), F32),
    )(idx, ew2, C, G, U)

    bw = 1024
    w_eff = pl.pallas_call(
        _weff_kernel,
        grid=(d_in // bw,),
        in_specs=[
            pl.BlockSpec((bw, d_out), lambda i: (i, 0)),
            pl.BlockSpec((bw, r), lambda i: (i, 0)),
            pl.BlockSpec((d_out, r), lambda i: (0, 0)),
            pl.BlockSpec((r, r), lambda i: (0, 0)),
        ],
        out_specs=pl.BlockSpec((bw, d_out), lambda i: (i, 0)),
        out_shape=jax.ShapeDtypeStruct((d_in, d_out), BF16),
        compiler_params=pltpu.CompilerParams(
            dimension_semantics=("parallel",)),
    )(W, U, U, m_eff)

    bn, bj = 1024, 512
    b2 = b.reshape(1, d_out)
    out = pl.pallas_call(
        _main_kernel,
        grid=(n // bn, d_out // bj),
        in_specs=[
            pl.BlockSpec((bn, d_in), lambda i, j: (i, 0)),
            pl.BlockSpec((d_in, bj), lambda i, j: (0, j)),
            pl.BlockSpec((1, bj), lambda i, j: (0, j)),
        ],
        out_specs=pl.BlockSpec((bn, bj), lambda i, j: (i, j)),
        out_shape=jax.ShapeDtypeStruct((n, d_out), F32),
        compiler_params=pltpu.CompilerParams(
            dimension_semantics=("parallel", "arbitrary")),
    )(x, w_eff, b2)
    return out


# speedup vs baseline: 1.1159x; 1.1139x over previous
import jax
import jax.numpy as jnp
from jax.experimental import pallas as pl
from jax.experimental.pallas import tpu as pltpu

F32 = jnp.float32
BF16 = jnp.bfloat16


def _calc_m_eff(idx, ew, c, g_all, u_all):
    g = g_all[idx]
    w = jnp.dot(ew, c, preferred_element_type=F32)
    p_dim, r, _ = g.shape
    m0 = jnp.zeros((r, r), dtype=F32)
    for p in range(p_dim):
        onehot = (jax.lax.broadcasted_iota(jnp.int32, (p_dim, 1), 0) == p)
        wp = jnp.dot(w, onehot.astype(F32), preferred_element_type=F32)
        m0 = m0 + wp * g[p]
    gn2 = jnp.sum(g * g)
    cn2 = jnp.sum(c * c)
    un2 = jnp.sum(u_all * u_all)
    scale = jax.lax.rsqrt(gn2) * jax.lax.rsqrt(cn2) / un2
    return m0 * scale


def _weff_kernel(idx_ref, ew_ref, c_ref, g_ref, uall_ref, w_ref, ui_ref,
                 o_ref):
    m_eff = _calc_m_eff(idx_ref[0], ew_ref[...], c_ref[...], g_ref,
                        uall_ref[...])
    a = jnp.dot(ui_ref[...], m_eff, preferred_element_type=F32)
    adapt = jax.lax.dot_general(
        a, uall_ref[...], (((1,), (1,)), ((), ())),
        preferred_element_type=F32)
    o_ref[...] = (w_ref[...] + adapt).astype(BF16)


def _main_kernel(x_ref, w_ref, b_ref, o_ref):
    bj = o_ref.shape[1]
    j = pl.program_id(1)
    xb = x_ref[...].astype(BF16)
    wj = w_ref[:, pl.ds(j * bj, bj)]
    o_ref[...] = (jnp.dot(xb, wj, preferred_element_type=F32)
                  + b_ref[...])


def kernel(x, tensor_idx, expert_weights, W, b, G, C, U):
    n, d_in = x.shape
    d_out = W.shape[1]
    k_dim, p_dim, r, _ = G.shape
    t_dim = expert_weights.shape[0]

    idx = jnp.asarray(tensor_idx, jnp.int32).reshape((1,))
    ew2 = expert_weights.reshape(1, t_dim).astype(F32)

    bw = 1024
    w_eff = pl.pallas_call(
        _weff_kernel,
        grid_spec=pltpu.PrefetchScalarGridSpec(
            num_scalar_prefetch=1,
            grid=(d_in // bw,),
            in_specs=[
                pl.BlockSpec((1, t_dim), lambda i, s: (0, 0)),
                pl.BlockSpec((t_dim, p_dim), lambda i, s: (0, 0)),
                pl.BlockSpec((k_dim, p_dim, r, r), lambda i, s: (0, 0, 0, 0)),
                pl.BlockSpec((d_out, r), lambda i, s: (0, 0)),
                pl.BlockSpec((bw, d_out), lambda i, s: (i, 0)),
                pl.BlockSpec((bw, r), lambda i, s: (i, 0)),
            ],
            out_specs=pl.BlockSpec((bw, d_out), lambda i, s: (i, 0)),
        ),
        out_shape=jax.ShapeDtypeStruct((d_in, d_out), BF16),
        compiler_params=pltpu.CompilerParams(
            dimension_semantics=("parallel",)),
    )(idx, ew2, C, G, U, W, U)

    bn, bj = 512, 2048
    b2 = b.reshape(1, d_out)
    out = pl.pallas_call(
        _main_kernel,
        grid=(n // bn, d_out // bj),
        in_specs=[
            pl.BlockSpec((bn, d_in), lambda i, j: (i, 0)),
            pl.BlockSpec((d_in, d_out), lambda i, j: (0, 0)),
            pl.BlockSpec((1, bj), lambda i, j: (0, j)),
        ],
        out_specs=pl.BlockSpec((bn, bj), lambda i, j: (i, j)),
        out_shape=jax.ShapeDtypeStruct((n, d_out), F32),
        compiler_params=pltpu.CompilerParams(
            dimension_semantics=("parallel", "arbitrary"),
            vmem_limit_bytes=63 * 1024 * 1024),
    )(x, w_eff, b2)
    return out
